# baseline (speedup 1.0000x reference)
import jax
import jax.numpy as jnp
from jax import lax
from jax.experimental import pallas as pl
from jax.experimental.pallas import tpu as pltpu
from jax.experimental.pallas import tpu_sc as plsc

NC = 2
NS = 16
NW = NC * NS
L = 16
WIN = 2048
CHUNK = 128
D = 15


def _make_table_update(n_rows, n_upd, own, emit_upd):
    own_last = n_rows - own * (NW - 1)
    diff = own - own_last
    assert own % L == 0 and own % 8 == 0
    assert own_last % L == 0 and own_last % 8 == 0 and diff % 8 == 0
    assert n_upd % WIN == 0
    n_win = n_upd // WIN

    out_type = [jax.ShapeDtypeStruct((n_rows, D), jnp.float32)]
    if emit_upd:
        out_type.append(jax.ShapeDtypeStruct((n_rows,), jnp.int32))

    mesh = plsc.VectorSubcoreMesh(
        core_axis_name="c", subcore_axis_name="s",
        num_cores=NC, num_subcores=NS)

    scratch = [
        pltpu.VMEM((WIN,), jnp.int32),
        pltpu.VMEM((own,), jnp.int32),
        pltpu.VMEM((n_upd + CHUNK,), jnp.int32),
        pltpu.VMEM((n_upd + CHUNK,), jnp.int32),
        pltpu.SemaphoreType.DMA,
        pltpu.SemaphoreType.DMA,
    ]

    def body(table, ids, feat, *rest):
        if emit_upd:
            out, upd = rest[0], rest[1]
            rest = rest[2:]
        else:
            out = rest[0]
            rest = rest[1:]
        idwin, winner, pos_list, id_list, csem, ssem = rest

        cid = lax.axis_index("c")
        sid = lax.axis_index("s")
        wid = sid * NC + cid
        base = wid * own
        own_sz = jnp.minimum(own, n_rows - base)
        iota = lax.broadcasted_iota(jnp.int32, (L,), 0)

        cp1 = pltpu.async_copy(
            table.at[pl.ds(base, own_last)], out.at[pl.ds(base, own_last)],
            csem)
        base2 = jnp.minimum(base + own_last, n_rows - diff)
        cp2 = pltpu.async_copy(
            table.at[pl.ds(base2, diff)], out.at[pl.ds(base2, diff)], csem)

        neg1 = jnp.full((L,), -1, jnp.int32)

        def memset_body(j, _):
            winner[pl.ds(j * L, L)] = neg1
            return 0
        lax.fori_loop(0, own_sz // L, memset_body, 0)

        def p1_window(w, _):
            pltpu.sync_copy(ids.at[pl.ds(w * WIN, WIN)], idwin)

            def p1_vreg(j, _):
                v = idwin[pl.ds(j * L, L)]
                rel = v - base
                m = (rel >= 0) & (rel < own_sz)
                rels = jnp.where(m, rel, 0)
                pos = w * WIN + j * L + iota
                plsc.store_scatter(winner, [rels], pos, mask=m)
                for _ in range(2):
                    g = plsc.load_gather(winner, [rels], mask=m)
                    fix = m & (g < pos)
                    plsc.store_scatter(winner, [rels], pos, mask=fix)
                return 0
            lax.fori_loop(0, WIN // L, p1_vreg, 0)
            return 0
        lax.fori_loop(0, n_win, p1_window, 0)

        def p2_window(w, off):
            pltpu.sync_copy(ids.at[pl.ds(w * WIN, WIN)], idwin)

            def p2_vreg(j, off):
                v = idwin[pl.ds(j * L, L)]
                rel = v - base
                m = (rel >= 0) & (rel < own_sz)
                rels = jnp.where(m, rel, 0)
                pos = w * WIN + j * L + iota
                g = plsc.load_gather(winner, [rels], mask=m)
                win = m & (g == pos)
                plsc.store_compressed(pos_list.at[pl.ds(off, L)], pos,
                                      mask=win)
                plsc.store_compressed(id_list.at[pl.ds(off, L)], v, mask=win)
                return off + jnp.max(plsc.all_reduce_population_count(win))
            return lax.fori_loop(0, WIN // L, p2_vreg, off)
        off = lax.fori_loop(0, n_win, p2_window, jnp.int32(0))

        if emit_upd:
            def upd_body(j, _):
                wv = winner[pl.ds(j * L, L)]
                winner[pl.ds(j * L, L)] = (wv >= 0).astype(jnp.int32)
                return 0
            lax.fori_loop(0, own_sz // L, upd_body, 0)
            pltpu.sync_copy(winner.at[pl.ds(0, own_last)],
                            upd.at[pl.ds(base, own_last)])
            s2 = base2 - base
            pltpu.sync_copy(winner.at[pl.ds(s2, diff)],
                            upd.at[pl.ds(base2, diff)])

        @pl.when(off > 0)
        def _():
            vfp = jnp.full((L,), pos_list[pl.ds(0, L)][0], jnp.int32)
            vfi = jnp.full((L,), id_list[pl.ds(0, L)][0], jnp.int32)
            for k in range(CHUNK // L):
                pos_list[pl.ds(off + k * L, L)] = vfp
                id_list[pl.ds(off + k * L, L)] = vfi

        cp1.wait()
        cp2.wait()
        ngrp = (off + L - 1) // L

        def grp_body(q, _):
            vp = pos_list[pl.ds(q * L, L)]
            vi = id_list[pl.ds(q * L, L)]
            descs = []
            for k in range(L):
                descs.append(pltpu.async_copy(
                    feat.at[pl.ds(vp[k], 1)], out.at[pl.ds(vi[k], 1)],
                    ssem))
            for d in descs:
                d.wait()
            return 0
        lax.fori_loop(0, ngrp, grp_body, 0)

    return pl.kernel(
        body, out_type=out_type, mesh=mesh, scratch_types=scratch,
        compiler_params=pltpu.CompilerParams(needs_layout_passes=False))


_node_update = _make_table_update(1000000, 32768, 31264, True)
_edge_update = _make_table_update(2000000, 16384, 62512, False)


def kernel(mem, src_ids, src_feat, dst_ids, dst_feat,
           edge_attr_mem, edge_idx, edge_feat):
    cat_ids = jnp.concatenate([src_ids.astype(jnp.int32),
                               dst_ids.astype(jnp.int32)])
    cat_feat = jnp.concatenate([src_feat, dst_feat], axis=0)
    new_mem, upd = _node_update(mem, cat_ids, cat_feat)
    new_edge, = _edge_update(edge_attr_mem, edge_idx.astype(jnp.int32),
                             edge_feat)
    return new_mem, new_edge, upd.astype(jnp.bool_)


# speedup vs baseline: 6.1854x; 6.1854x over previous
import jax
import jax.numpy as jnp
from jax import lax
from jax.experimental import pallas as pl
from jax.experimental.pallas import tpu as pltpu
from jax.experimental.pallas import tpu_sc as plsc

NC = 2
NS = 16
NW = NC * NS
L = 16
WIN = 2048
CHUNK = 128
D = 15


def _make_table_update(n_rows, n_upd, own, emit_upd):
    own_last = n_rows - own * (NW - 1)
    diff = own - own_last
    assert own % L == 0 and own % 8 == 0
    assert own_last % L == 0 and own_last % 8 == 0 and diff % 8 == 0
    assert n_upd % WIN == 0
    n_win = n_upd // WIN

    out_type = [jax.ShapeDtypeStruct((n_rows, D), jnp.float32)]
    if emit_upd:
        out_type.append(jax.ShapeDtypeStruct((n_rows,), jnp.int32))

    mesh = plsc.VectorSubcoreMesh(
        core_axis_name="c", subcore_axis_name="s",
        num_cores=NC, num_subcores=NS)

    scratch = [
        pltpu.VMEM((WIN,), jnp.int32),
        pltpu.VMEM((own,), jnp.int32),
        pltpu.VMEM((n_upd + CHUNK,), jnp.int32),
        pltpu.VMEM((n_upd + CHUNK,), jnp.int32),
        pltpu.SemaphoreType.DMA,
        pltpu.SemaphoreType.DMA,
    ]

    def body(table, ids, feat, *rest):
        if emit_upd:
            out, upd = rest[0], rest[1]
            rest = rest[2:]
        else:
            out = rest[0]
            rest = rest[1:]
        idwin, winner, pos_list, id_list, csem, ssem = rest

        cid = lax.axis_index("c")
        sid = lax.axis_index("s")
        wid = sid * NC + cid
        base = wid * own
        own_sz = jnp.minimum(own, n_rows - base)
        iota = lax.broadcasted_iota(jnp.int32, (L,), 0)

        cp1 = pltpu.async_copy(
            table.at[pl.ds(base, own_last)], out.at[pl.ds(base, own_last)],
            csem)
        base2 = jnp.minimum(base + own_last, n_rows - diff)
        cp2 = pltpu.async_copy(
            table.at[pl.ds(base2, diff)], out.at[pl.ds(base2, diff)], csem)

        neg1 = jnp.full((L,), -1, jnp.int32)

        def memset_body(j, _):
            winner[pl.ds(j * L, L)] = neg1
            return 0
        lax.fori_loop(0, own_sz // L, memset_body, 0)

        def p1_window(w, _):
            pltpu.sync_copy(ids.at[pl.ds(w * WIN, WIN)], idwin)

            def p1_vreg(j, _):
                v = idwin[pl.ds(j * L, L)]
                rel = v - base
                m = (rel >= 0) & (rel < own_sz)
                rels = jnp.where(m, rel, 0)
                pos = w * WIN + j * L + iota
                plsc.store_scatter(winner, [rels], pos, mask=m)
                for _ in range(2):
                    g = plsc.load_gather(winner, [rels], mask=m)
                    fix = m & (g < pos)
                    plsc.store_scatter(winner, [rels], pos, mask=fix)
                return 0
            lax.fori_loop(0, WIN // L, p1_vreg, 0)
            return 0
        lax.fori_loop(0, n_win, p1_window, 0)

        def p2_window(w, off):
            pltpu.sync_copy(ids.at[pl.ds(w * WIN, WIN)], idwin)

            def p2_vreg(j, off):
                v = idwin[pl.ds(j * L, L)]
                rel = v - base
                m = (rel >= 0) & (rel < own_sz)
                rels = jnp.where(m, rel, 0)
                pos = w * WIN + j * L + iota
                g = plsc.load_gather(winner, [rels], mask=m)
                win = m & (g == pos)
                plsc.store_compressed(pos_list.at[pl.ds(off, L)], pos,
                                      mask=win)
                plsc.store_compressed(id_list.at[pl.ds(off, L)], v, mask=win)
                return off + jnp.max(plsc.all_reduce_population_count(win))
            return lax.fori_loop(0, WIN // L, p2_vreg, off)
        off = lax.fori_loop(0, n_win, p2_window, jnp.int32(0))

        if emit_upd:
            def upd_body(j, _):
                wv = winner[pl.ds(j * L, L)]
                winner[pl.ds(j * L, L)] = (wv >= 0).astype(jnp.int32)
                return 0
            lax.fori_loop(0, own_sz // L, upd_body, 0)
            pltpu.sync_copy(winner.at[pl.ds(0, own_last)],
                            upd.at[pl.ds(base, own_last)])
            s2 = base2 - base
            pltpu.sync_copy(winner.at[pl.ds(s2, diff)],
                            upd.at[pl.ds(base2, diff)])

        @pl.when(off > 0)
        def _():
            vfp = jnp.full((L,), pos_list[pl.ds(0, L)][0], jnp.int32)
            vfi = jnp.full((L,), id_list[pl.ds(0, L)][0], jnp.int32)
            for k in range(CHUNK // L):
                pos_list[pl.ds(off + k * L, L)] = vfp
                id_list[pl.ds(off + k * L, L)] = vfi

        cp1.wait()
        cp2.wait()
        ngrp = (off + L - 1) // L

        def grp_body(q, _):
            vp = pos_list[pl.ds(q * L, L)]
            vi = id_list[pl.ds(q * L, L)]
            descs = []
            for k in range(L):
                descs.append(pltpu.async_copy(
                    feat.at[pl.ds(vp[k], 1)], out.at[pl.ds(vi[k], 1)],
                    ssem))
            for d in descs:
                d.wait()
            return 0
        lax.fori_loop(0, ngrp, grp_body, 0)

    return pl.kernel(
        body, out_type=out_type, mesh=mesh, scratch_types=scratch,
        compiler_params=pltpu.CompilerParams(needs_layout_passes=False,
                                             use_tc_tiling_on_sc=False))


_node_update = _make_table_update(1000000, 32768, 31264, True)
_edge_update = _make_table_update(2000000, 16384, 62512, False)


def kernel(mem, src_ids, src_feat, dst_ids, dst_feat,
           edge_attr_mem, edge_idx, edge_feat):
    cat_ids = jnp.concatenate([src_ids.astype(jnp.int32),
                               dst_ids.astype(jnp.int32)])
    cat_feat = jnp.concatenate([src_feat, dst_feat], axis=0)
    new_mem, upd = _node_update(mem, cat_ids, cat_feat)
    new_edge, = _edge_update(edge_attr_mem, edge_idx.astype(jnp.int32),
                             edge_feat)
    return new_mem, new_edge, upd.astype(jnp.bool_)


# speedup vs baseline: 13.9409x; 2.2538x over previous
import jax
import jax.numpy as jnp
from jax import lax
from jax.experimental import pallas as pl
from jax.experimental.pallas import tpu as pltpu
from jax.experimental.pallas import tpu_sc as plsc

NC = 2
NS = 16
NW = NC * NS
L = 16
WIN = 2048
CHUNK = 128
CH = 640
D = 15


def _make_table_update(n_rows, n_upd, own, emit_upd):
    own_last = n_rows - own * (NW - 1)
    diff = own - own_last
    assert own % L == 0 and own % 8 == 0
    assert own_last % L == 0 and own_last % 8 == 0 and diff % 8 == 0
    assert n_upd % WIN == 0
    n_win = n_upd // WIN

    out_type = [jax.ShapeDtypeStruct((n_rows, D), jnp.float32)]
    if emit_upd:
        out_type.append(jax.ShapeDtypeStruct((n_rows,), jnp.int32))

    mesh = plsc.VectorSubcoreMesh(
        core_axis_name="c", subcore_axis_name="s",
        num_cores=NC, num_subcores=NS)

    scratch = [
        pltpu.VMEM((WIN,), jnp.int32),
        pltpu.VMEM((own,), jnp.int32),
        pltpu.VMEM((n_upd + CHUNK,), jnp.int32),
        pltpu.VMEM((n_upd + CHUNK,), jnp.int32),
        pltpu.VMEM((CH, D), jnp.float32),
        pltpu.VMEM((CH, D), jnp.float32),
        pltpu.SemaphoreType.DMA,
        pltpu.SemaphoreType.DMA,
        pltpu.SemaphoreType.DMA,
    ]

    def body(table, ids, feat, *rest):
        if emit_upd:
            out, upd = rest[0], rest[1]
            rest = rest[2:]
        else:
            out = rest[0]
            rest = rest[1:]
        (idwin, winner, pos_list, id_list, cb0, cb1,
         cgsem, cssem, ssem) = rest

        cid = lax.axis_index("c")
        sid = lax.axis_index("s")
        wid = sid * NC + cid
        base = wid * own
        own_sz = jnp.minimum(own, n_rows - base)
        iota = lax.broadcasted_iota(jnp.int32, (L,), 0)

        last_start = base + own_sz - CH

        def copy_pair(p, _):
            st0 = jnp.minimum(base + (2 * p) * CH, last_start)
            st1 = jnp.minimum(base + (2 * p + 1) * CH, last_start)
            g0 = pltpu.async_copy(table.at[pl.ds(st0, CH)], cb0, cgsem)
            g1 = pltpu.async_copy(table.at[pl.ds(st1, CH)], cb1, cgsem)
            g0.wait()
            s0 = pltpu.async_copy(cb0, out.at[pl.ds(st0, CH)], cssem)
            g1.wait()
            s1 = pltpu.async_copy(cb1, out.at[pl.ds(st1, CH)], cssem)
            s0.wait()
            s1.wait()
            return 0
        npair = (own_sz + 2 * CH - 1) // (2 * CH)
        lax.fori_loop(0, npair, copy_pair, 0)

        neg1 = jnp.full((L,), -1, jnp.int32)

        def memset_body(j, _):
            winner[pl.ds(j * L, L)] = neg1
            return 0
        lax.fori_loop(0, own_sz // L, memset_body, 0)

        def p1_window(w, _):
            pltpu.sync_copy(ids.at[pl.ds(w * WIN, WIN)], idwin)

            def p1_vreg(j, _):
                v = idwin[pl.ds(j * L, L)]
                rel = v - base
                m = (rel >= 0) & (rel < own_sz)
                rels = jnp.where(m, rel, 0)
                pos = w * WIN + j * L + iota
                plsc.store_scatter(winner, [rels], pos, mask=m)
                for _ in range(2):
                    g = plsc.load_gather(winner, [rels], mask=m)
                    fix = m & (g < pos)
                    plsc.store_scatter(winner, [rels], pos, mask=fix)
                return 0
            lax.fori_loop(0, WIN // L, p1_vreg, 0)
            return 0
        lax.fori_loop(0, n_win, p1_window, 0)

        def p2_window(w, off):
            pltpu.sync_copy(ids.at[pl.ds(w * WIN, WIN)], idwin)

            def p2_vreg(j, off):
                v = idwin[pl.ds(j * L, L)]
                rel = v - base
                m = (rel >= 0) & (rel < own_sz)
                rels = jnp.where(m, rel, 0)
                pos = w * WIN + j * L + iota
                g = plsc.load_gather(winner, [rels], mask=m)
                win = m & (g == pos)
                plsc.store_compressed(pos_list.at[pl.ds(off, L)], pos,
                                      mask=win)
                plsc.store_compressed(id_list.at[pl.ds(off, L)], v, mask=win)
                return off + jnp.max(plsc.all_reduce_population_count(win))
            return lax.fori_loop(0, WIN // L, p2_vreg, off)
        off = lax.fori_loop(0, n_win, p2_window, jnp.int32(0))

        if emit_upd:
            def upd_body(j, _):
                wv = winner[pl.ds(j * L, L)]
                winner[pl.ds(j * L, L)] = (wv >= 0).astype(jnp.int32)
                return 0
            lax.fori_loop(0, own_sz // L, upd_body, 0)
            pltpu.sync_copy(winner.at[pl.ds(0, own_last)],
                            upd.at[pl.ds(base, own_last)])
            s2 = jnp.minimum(own_last, own_sz - diff)
            pltpu.sync_copy(winner.at[pl.ds(s2, diff)],
                            upd.at[pl.ds(base + s2, diff)])

        @pl.when(off > 0)
        def _():
            vfp = jnp.full((L,), pos_list[pl.ds(0, L)][0], jnp.int32)
            vfi = jnp.full((L,), id_list[pl.ds(0, L)][0], jnp.int32)
            for k in range(CHUNK // L):
                pos_list[pl.ds(off + k * L, L)] = vfp
                id_list[pl.ds(off + k * L, L)] = vfi

        ngrp = (off + L - 1) // L

        def grp_body(q, _):
            vp = pos_list[pl.ds(q * L, L)]
            vi = id_list[pl.ds(q * L, L)]
            descs = []
            for k in range(L):
                descs.append(pltpu.async_copy(
                    feat.at[pl.ds(vp[k], 1)], out.at[pl.ds(vi[k], 1)],
                    ssem))
            for d in descs:
                d.wait()
            return 0
        lax.fori_loop(0, ngrp, grp_body, 0)

    return pl.kernel(
        body, out_type=out_type, mesh=mesh, scratch_types=scratch,
        compiler_params=pltpu.CompilerParams(needs_layout_passes=False,
                                             use_tc_tiling_on_sc=False))


_node_update = _make_table_update(1000000, 32768, 31264, True)
_edge_update = _make_table_update(2000000, 16384, 62512, False)


def kernel(mem, src_ids, src_feat, dst_ids, dst_feat,
           edge_attr_mem, edge_idx, edge_feat):
    cat_ids = jnp.concatenate([src_ids.astype(jnp.int32),
                               dst_ids.astype(jnp.int32)])
    cat_feat = jnp.concatenate([src_feat, dst_feat], axis=0)
    new_mem, upd = _node_update(mem, cat_ids, cat_feat)
    new_edge, = _edge_update(edge_attr_mem, edge_idx.astype(jnp.int32),
                             edge_feat)
    return new_mem, new_edge, upd.astype(jnp.bool_)


# speedup vs baseline: 26.8568x; 1.9265x over previous
import jax
import jax.numpy as jnp
from jax import lax
from jax.experimental import pallas as pl
from jax.experimental.pallas import tpu as pltpu
from jax.experimental.pallas import tpu_sc as plsc

NC = 2
NS = 16
NW = NC * NS
L = 16
WIN = 2048
CHUNK = 128
D = 15


def _make_table_update(n_rows, n_upd, own, emit_upd):
    own_last = n_rows - own * (NW - 1)
    diff = own - own_last
    assert own % L == 0 and own % 8 == 0
    assert own_last % L == 0 and own_last % 8 == 0 and diff % 8 == 0
    assert n_upd % WIN == 0
    n_win = n_upd // WIN

    out_type = []
    if emit_upd:
        out_type = [jax.ShapeDtypeStruct((n_rows,), jnp.int32)]

    mesh = plsc.VectorSubcoreMesh(
        core_axis_name="c", subcore_axis_name="s",
        num_cores=NC, num_subcores=NS)

    scratch = [
        pltpu.VMEM((WIN,), jnp.int32),
        pltpu.VMEM((own,), jnp.int32),
        pltpu.VMEM((n_upd + CHUNK,), jnp.int32),
        pltpu.VMEM((n_upd + CHUNK,), jnp.int32),
        pltpu.SemaphoreType.DMA,
    ]

    def body(table, ids, feat, *rest):
        if emit_upd:
            upd = rest[0]
            rest = rest[1:]
        idwin, winner, pos_list, id_list, ssem = rest

        cid = lax.axis_index("c")
        sid = lax.axis_index("s")
        wid = sid * NC + cid
        base = wid * own
        own_sz = jnp.minimum(own, n_rows - base)
        iota = lax.broadcasted_iota(jnp.int32, (L,), 0)

        neg1 = jnp.full((L,), -1, jnp.int32)

        def memset_body(j, _):
            winner[pl.ds(j * L, L)] = neg1
            return 0
        lax.fori_loop(0, own_sz // L, memset_body, 0)

        def p1_window(w, _):
            pltpu.sync_copy(ids.at[pl.ds(w * WIN, WIN)], idwin)

            def p1_vreg(j, _):
                v = idwin[pl.ds(j * L, L)]
                rel = v - base
                m = (rel >= 0) & (rel < own_sz)
                rels = jnp.where(m, rel, 0)
                pos = w * WIN + j * L + iota
                plsc.store_scatter(winner, [rels], pos, mask=m)
                for _ in range(2):
                    g = plsc.load_gather(winner, [rels], mask=m)
                    fix = m & (g < pos)
                    plsc.store_scatter(winner, [rels], pos, mask=fix)
                return 0
            lax.fori_loop(0, WIN // L, p1_vreg, 0)
            return 0
        lax.fori_loop(0, n_win, p1_window, 0)

        def p2_window(w, off):
            pltpu.sync_copy(ids.at[pl.ds(w * WIN, WIN)], idwin)

            def p2_vreg(j, off):
                v = idwin[pl.ds(j * L, L)]
                rel = v - base
                m = (rel >= 0) & (rel < own_sz)
                rels = jnp.where(m, rel, 0)
                pos = w * WIN + j * L + iota
                g = plsc.load_gather(winner, [rels], mask=m)
                win = m & (g == pos)
                plsc.store_compressed(pos_list.at[pl.ds(off, L)], pos,
                                      mask=win)
                plsc.store_compressed(id_list.at[pl.ds(off, L)], v, mask=win)
                return off + jnp.max(plsc.all_reduce_population_count(win))
            return lax.fori_loop(0, WIN // L, p2_vreg, off)
        off = lax.fori_loop(0, n_win, p2_window, jnp.int32(0))

        if emit_upd:
            def upd_body(j, _):
                wv = winner[pl.ds(j * L, L)]
                winner[pl.ds(j * L, L)] = (wv >= 0).astype(jnp.int32)
                return 0
            lax.fori_loop(0, own_sz // L, upd_body, 0)
            pltpu.sync_copy(winner.at[pl.ds(0, own_last)],
                            upd.at[pl.ds(base, own_last)])
            s2 = jnp.minimum(own_last, own_sz - diff)
            pltpu.sync_copy(winner.at[pl.ds(s2, diff)],
                            upd.at[pl.ds(base + s2, diff)])

        @pl.when(off > 0)
        def _():
            vfp = jnp.full((L,), pos_list[pl.ds(0, L)][0], jnp.int32)
            vfi = jnp.full((L,), id_list[pl.ds(0, L)][0], jnp.int32)
            for k in range(CHUNK // L):
                pos_list[pl.ds(off + k * L, L)] = vfp
                id_list[pl.ds(off + k * L, L)] = vfi

        ngrp = (off + L - 1) // L

        def grp_body(q, _):
            vp = pos_list[pl.ds(q * L, L)]
            vi = id_list[pl.ds(q * L, L)]
            descs = []
            for k in range(L):
                descs.append(pltpu.async_copy(
                    feat.at[pl.ds(vp[k], 1)], table.at[pl.ds(vi[k], 1)],
                    ssem))
            for d in descs:
                d.wait()
            return 0
        lax.fori_loop(0, ngrp, grp_body, 0)

    return pl.kernel(
        body, out_type=out_type, mesh=mesh, scratch_types=scratch,
        compiler_params=pltpu.CompilerParams(needs_layout_passes=False))


_node_update = _make_table_update(1000000, 32768, 31264, True)
_edge_update = _make_table_update(2000000, 16384, 62512, False)


def kernel(mem, src_ids, src_feat, dst_ids, dst_feat,
           edge_attr_mem, edge_idx, edge_feat):
    cat_ids = jnp.concatenate([src_ids.astype(jnp.int32),
                               dst_ids.astype(jnp.int32)])
    cat_feat = jnp.concatenate([src_feat, dst_feat], axis=0)
    mem_ref = jax.new_ref(mem)
    edge_ref = jax.new_ref(edge_attr_mem)
    upd = _node_update(mem_ref, cat_ids, cat_feat)[0]
    _edge_update(edge_ref, edge_idx.astype(jnp.int32), edge_feat)
    return mem_ref[...], edge_ref[...], upd.astype(jnp.bool_)


# speedup vs baseline: 28.5685x; 1.0637x over previous
import jax
import jax.numpy as jnp
from jax import lax
from jax.experimental import pallas as pl
from jax.experimental.pallas import tpu as pltpu
from jax.experimental.pallas import tpu_sc as plsc

NC = 2
NS = 16
NW = NC * NS
L = 16
WIN = 2048
CHUNK = 128
D = 15


def _make_table_update(n_rows, n_upd, own, emit_upd):
    own_last = n_rows - own * (NW - 1)
    diff = own - own_last
    assert own % L == 0 and own % 8 == 0
    assert own_last % L == 0 and own_last % 8 == 0 and diff % 8 == 0
    assert n_upd % WIN == 0
    n_win = n_upd // WIN

    out_type = []
    if emit_upd:
        out_type = [jax.ShapeDtypeStruct((n_rows,), jnp.int32)]

    mesh = plsc.VectorSubcoreMesh(
        core_axis_name="c", subcore_axis_name="s",
        num_cores=NC, num_subcores=NS)

    scratch = [
        pltpu.VMEM((WIN,), jnp.int32),
        pltpu.VMEM((own,), jnp.int32),
        pltpu.VMEM((n_upd + CHUNK,), jnp.int32),
        pltpu.VMEM((n_upd + CHUNK,), jnp.int32),
        pltpu.SemaphoreType.DMA,
    ]

    def body(table, ids, feat, *rest):
        if emit_upd:
            upd = rest[0]
            rest = rest[1:]
        idwin, winner, pos_list, id_list, ssem = rest

        cid = lax.axis_index("c")
        sid = lax.axis_index("s")
        wid = sid * NC + cid
        base = wid * own
        own_sz = jnp.minimum(own, n_rows - base)
        iota = lax.broadcasted_iota(jnp.int32, (L,), 0)

        neg1 = jnp.full((L,), -1, jnp.int32)

        def memset_body(j, _):
            winner[pl.ds(j * L, L)] = neg1
            return 0
        lax.fori_loop(0, own_sz // L, memset_body, 0)

        def p1_window(w, off):
            pltpu.sync_copy(ids.at[pl.ds(w * WIN, WIN)], idwin)

            def p1_vreg(j, off):
                v = idwin[pl.ds(j * L, L)]
                rel = v - base
                m = (rel >= 0) & (rel < own_sz)
                rels = jnp.where(m, rel, 0)
                pos = w * WIN + j * L + iota
                plsc.store_scatter(winner, [rels], pos, mask=m)
                plsc.store_compressed(pos_list.at[pl.ds(off, L)], pos,
                                      mask=m)
                plsc.store_compressed(id_list.at[pl.ds(off, L)], v, mask=m)
                return off + jnp.max(plsc.all_reduce_population_count(m))
            return lax.fori_loop(0, WIN // L, p1_vreg, off)
        cand = lax.fori_loop(0, n_win, p1_window, jnp.int32(0))
        ncv = (cand + L - 1) // L

        @pl.when(cand > 0)
        def _():
            vfp = jnp.full((L,), pos_list[pl.ds(0, L)][0], jnp.int32)
            vfi = jnp.full((L,), id_list[pl.ds(0, L)][0], jnp.int32)
            pos_list[pl.ds(cand, L)] = vfp
            id_list[pl.ds(cand, L)] = vfi

        def fix_round(q, _):
            vp = pos_list[pl.ds(q * L, L)]
            vi = id_list[pl.ds(q * L, L)]
            rel = vi - base
            g = plsc.load_gather(winner, [rel])
            fix = g < vp
            plsc.store_scatter(winner, [rel], vp, mask=fix)
            return 0
        lax.fori_loop(0, ncv, fix_round, 0)
        lax.fori_loop(0, ncv, fix_round, 0)

        def sel_vreg(q, off):
            vp = pos_list[pl.ds(q * L, L)]
            vi = id_list[pl.ds(q * L, L)]
            rel = vi - base
            g = plsc.load_gather(winner, [rel])
            win = g == vp
            plsc.store_compressed(pos_list.at[pl.ds(off, L)], vp, mask=win)
            plsc.store_compressed(id_list.at[pl.ds(off, L)], vi, mask=win)
            return off + jnp.max(plsc.all_reduce_population_count(win))
        off = lax.fori_loop(0, ncv, sel_vreg, jnp.int32(0))

        if emit_upd:
            def upd_body(j, _):
                wv = winner[pl.ds(j * L, L)]
                winner[pl.ds(j * L, L)] = (wv >= 0).astype(jnp.int32)
                return 0
            lax.fori_loop(0, own_sz // L, upd_body, 0)
            pltpu.sync_copy(winner.at[pl.ds(0, own_last)],
                            upd.at[pl.ds(base, own_last)])
            s2 = jnp.minimum(own_last, own_sz - diff)
            pltpu.sync_copy(winner.at[pl.ds(s2, diff)],
                            upd.at[pl.ds(base + s2, diff)])

        @pl.when(off > 0)
        def _():
            vfp = jnp.full((L,), pos_list[pl.ds(0, L)][0], jnp.int32)
            vfi = jnp.full((L,), id_list[pl.ds(0, L)][0], jnp.int32)
            for k in range(CHUNK // L):
                pos_list[pl.ds(off + k * L, L)] = vfp
                id_list[pl.ds(off + k * L, L)] = vfi

        ngrp = (off + L - 1) // L

        def grp_body(q, _):
            vp = pos_list[pl.ds(q * L, L)]
            vi = id_list[pl.ds(q * L, L)]
            descs = []
            for k in range(L):
                descs.append(pltpu.async_copy(
                    feat.at[pl.ds(vp[k], 1)], table.at[pl.ds(vi[k], 1)],
                    ssem))
            for d in descs:
                d.wait()
            return 0
        lax.fori_loop(0, ngrp, grp_body, 0)

    return pl.kernel(
        body, out_type=out_type, mesh=mesh, scratch_types=scratch,
        compiler_params=pltpu.CompilerParams(needs_layout_passes=False))


_node_update = _make_table_update(1000000, 32768, 31264, True)
_edge_update = _make_table_update(2000000, 16384, 62512, False)


def kernel(mem, src_ids, src_feat, dst_ids, dst_feat,
           edge_attr_mem, edge_idx, edge_feat):
    cat_ids = jnp.concatenate([src_ids.astype(jnp.int32),
                               dst_ids.astype(jnp.int32)])
    cat_feat = jnp.concatenate([src_feat, dst_feat], axis=0)
    mem_ref = jax.new_ref(mem)
    edge_ref = jax.new_ref(edge_attr_mem)
    upd = _node_update(mem_ref, cat_ids, cat_feat)[0]
    _edge_update(edge_ref, edge_idx.astype(jnp.int32), edge_feat)
    return mem_ref[...], edge_ref[...], upd.astype(jnp.bool_)
